# baseline (speedup 1.0000x reference)
import functools
import math

import jax
import jax.numpy as jnp
import numpy as np
from jax import lax
from jax.experimental import pallas as pl
from jax.experimental.pallas import tpu as pltpu
from jax.experimental.pallas import tpu_sc as plsc


def _pe_const(max_len, emb):
    pe = np.zeros((max_len, emb), dtype=np.float32)
    position = np.arange(0, max_len, dtype=np.float32)[:, None]
    div_term = np.exp(
        np.arange(0, emb, 2, dtype=np.float32) * -(math.log(10000.0) / emb))
    pe[:, 0::2] = np.sin(position * div_term)
    pe[:, 1::2] = np.cos(position * div_term)
    return pe


@functools.lru_cache(maxsize=None)
def _make_sc_kernel(S, B, D):
    info = plsc.get_sparse_core_info()
    NC, NS, L = info.num_cores, info.num_subcores, info.num_lanes
    NW = NC * NS
    CB = B // NW
    NV = D // L
    RU = 8
    SCALE = float(np.sqrt(D))

    assert B % NW == 0 and D % L == 0 and CB % RU == 0 and S % 2 == 0

    mesh = plsc.VectorSubcoreMesh(core_axis_name="c", subcore_axis_name="s")

    @functools.partial(
        pl.kernel,
        mesh=mesh,
        compiler_params=pltpu.CompilerParams(use_tc_tiling_on_sc=False),
        out_type=jax.ShapeDtypeStruct((S, B, D), jnp.float32),
        scratch_types=[
            pltpu.VMEM((S, CB), jnp.int32),
            pltpu.VMEM((S, D), jnp.float32),
            pltpu.VMEM((CB, D), jnp.float32),
            pltpu.VMEM((CB, D), jnp.float32),
            pltpu.VMEM((CB, D), jnp.float32),
            pltpu.VMEM((CB, D), jnp.float32),
            pltpu.SemaphoreType.DMA,
            pltpu.SemaphoreType.DMA,
            pltpu.SemaphoreType.DMA,
            pltpu.SemaphoreType.DMA,
        ],
    )
    def k(src_hbm, table_hbm, pe_hbm, out_hbm,
          idx_v, pe_v, in0, in1, o0, o1, g0, g1, w0, w1):
        wid = lax.axis_index("s") * NC + lax.axis_index("c")
        col0 = wid * CB
        ins = (in0, in1)
        outs = (o0, o1)
        gsems = (g0, g1)
        osems = (w0, w1)

        pltpu.sync_copy(src_hbm.at[:, pl.ds(col0, CB)], idx_v)
        pltpu.sync_copy(pe_hbm, pe_v)

        pltpu.async_copy(table_hbm.at[idx_v.at[0]], in0, g0)
        pltpu.async_copy(table_hbm.at[idx_v.at[1]], in1, g1)

        def halfstep(i, par):
            s = 2 * i + par
            inb, outb = ins[par], outs[par]
            gs, osem = gsems[par], osems[par]
            pltpu.make_async_copy(table_hbm.at[pl.ds(0, CB)], inb, gs).wait()
            @pl.when(i > 0)
            def _():
                pltpu.make_async_copy(
                    outb, out_hbm.at[0, pl.ds(col0, CB)], osem).wait()

            pe_regs = [pe_v[s, pl.ds(j * L, L)] for j in range(NV)]

            def rows(r0, carry):
                base = r0 * RU
                for u in range(RU):
                    r = base + u
                    for j in range(NV):
                        outb[r, pl.ds(j * L, L)] = (
                            inb[r, pl.ds(j * L, L)] * SCALE + pe_regs[j])
                return carry

            lax.fori_loop(0, CB // RU, rows, 0, unroll=False)

            pltpu.async_copy(outb, out_hbm.at[s, pl.ds(col0, CB)], osem)
            @pl.when(s + 2 < S)
            def _():
                pltpu.async_copy(table_hbm.at[idx_v.at[s + 2]], inb, gs)

        def step(i, carry):
            halfstep(i, 0)
            halfstep(i, 1)
            return carry

        lax.fori_loop(0, S // 2, step, 0, unroll=False)

        pltpu.make_async_copy(o0, out_hbm.at[0, pl.ds(col0, CB)], w0).wait()
        pltpu.make_async_copy(o1, out_hbm.at[0, pl.ds(col0, CB)], w1).wait()

    return k


def kernel(src, table, step):
    del step
    S, B = src.shape
    V, D = table.shape
    pe = jnp.asarray(_pe_const(S, D))
    return _make_sc_kernel(S, B, D)(src.astype(jnp.int32), table, pe)
